# Initial kernel scaffold; baseline (speedup 1.0000x reference)
import jax
import jax.numpy as jnp
from jax.experimental import pallas as pl


def kernel(input_ids, word_table, pos_table, tok_table, gamma, beta):
    raise NotImplementedError("write your pallas kernel here")


# speedup vs baseline: 2.9196x; 2.9196x over previous
import functools

import jax
import jax.numpy as jnp
from jax import lax
from jax.experimental import pallas as pl
from jax.experimental.pallas import tpu as pltpu
from jax.experimental.pallas import tpu_sc as plsc

EPS = 1e-12
LANES = 16


def _rsqrt16(x):
    v = jnp.full((LANES,), x, dtype=jnp.float32)
    i = plsc.bitcast(v, jnp.int32)
    i = jnp.int32(0x5F3759DF) - lax.shift_right_logical(i, 1)
    r = plsc.bitcast(i, jnp.float32)
    for _ in range(3):
        r = r * (1.5 - 0.5 * v * r * r)
    return r


def kernel(input_ids, word_table, pos_table, tok_table, gamma, beta):
    B, L = input_ids.shape
    V, H = word_table.shape
    NW = 32
    RPW = B // NW
    HALF = L // 2
    NK = H // LANES

    ids = input_ids.astype(jnp.int32).reshape(B, 2, HALF)
    mesh = plsc.VectorSubcoreMesh(core_axis_name="c", subcore_axis_name="s")

    @functools.partial(
        pl.kernel,
        out_type=jax.ShapeDtypeStruct((B, L, H), jnp.float32),
        mesh=mesh,
        compiler_params=pltpu.CompilerParams(needs_layout_passes=False),
        scratch_types=[
            pltpu.VMEM((2, HALF), jnp.int32),
            pltpu.VMEM((L, H), jnp.float32),
            pltpu.VMEM((L, H), jnp.float32),
            pltpu.VMEM((L, H), jnp.float32),
            pltpu.VMEM((H,), jnp.float32),
            pltpu.VMEM((H,), jnp.float32),
            pltpu.VMEM((H,), jnp.float32),
            pltpu.SemaphoreType.DMA,
        ],
    )
    def sc_fn(ids_h, wt_h, pos_h, tok_h, g_h, b_h, out_h,
              idx_v, buf_v, obuf_v, bias_v, tok_v, g_v, b_v, sem):
        cid = lax.axis_index("c")
        sid = lax.axis_index("s")
        wid = sid * 2 + cid

        pltpu.sync_copy(g_h, g_v)
        pltpu.sync_copy(b_h, b_v)
        pltpu.sync_copy(tok_h.at[0], tok_v)
        pltpu.sync_copy(pos_h.at[pl.ds(0, L)], bias_v)

        @plsc.parallel_loop(0, L)
        def _(t):
            for k in range(NK):
                s = pl.ds(k * LANES, LANES)
                bias_v[t, s] = bias_v[t, s] + tok_v[s]

        def row_body(r, carry):
            row = wid * RPW + r
            pltpu.sync_copy(ids_h.at[row], idx_v)
            cp0 = pltpu.async_copy(
                wt_h.at[idx_v.at[0]], buf_v.at[pl.ds(0, HALF)], sem)
            cp1 = pltpu.async_copy(
                wt_h.at[idx_v.at[1]], buf_v.at[pl.ds(HALF, HALF)], sem)
            cp0.wait()
            cp1.wait()

            @plsc.parallel_loop(0, L, unroll=2)
            def _(j):
                ys = []
                for k in range(NK):
                    s = pl.ds(k * LANES, LANES)
                    ys.append(buf_v[j, s] + bias_v[j, s])
                t4 = (((ys[0] + ys[1]) + (ys[2] + ys[3]))
                      + ((ys[4] + ys[5]) + (ys[6] + ys[7])))
                ssum = plsc.cumsum(t4)[LANES - 1]
                sqs = [y * y for y in ys]
                q4 = (((sqs[0] + sqs[1]) + (sqs[2] + sqs[3]))
                      + ((sqs[4] + sqs[5]) + (sqs[6] + sqs[7])))
                ssq = plsc.cumsum(q4)[LANES - 1]
                mean = ssum * (1.0 / H)
                var = ssq * (1.0 / H) - mean * mean
                inv = _rsqrt16(var + EPS)
                for k in range(NK):
                    s = pl.ds(k * LANES, LANES)
                    obuf_v[j, s] = (ys[k] - mean) * (inv * g_v[s]) + b_v[s]

            pltpu.sync_copy(obuf_v, out_h.at[row])
            return carry

        lax.fori_loop(0, RPW, row_body, 0)

    return sc_fn(ids, word_table, pos_table, tok_table, gamma, beta)
